# Initial kernel scaffold; baseline (speedup 1.0000x reference)
import jax
import jax.numpy as jnp
from jax.experimental import pallas as pl


def kernel(x, edge_index, batch, denoising_label, W_g1, b_g1, ln1_g, ln1_b, W_g2, b_g2, ln2_g, ln2_b, cls_ln0_g, cls_ln0_b, cls_W1, cls_b1, cls_ln1_g, cls_ln1_b, cls_W2, cls_b2, cls_ln2_g, cls_ln2_b, cls_W3, cls_b3):
    raise NotImplementedError("write your pallas kernel here")


# speedup vs baseline: 20.8820x; 20.8820x over previous
import functools

import jax
import jax.numpy as jnp
from jax import lax
from jax.experimental import pallas as pl
from jax.experimental.pallas import tpu as pltpu
from jax.experimental.pallas import tpu_sc as plsc

N = 10000
E = 320000
D = 128
DH = D // 2
NC = 2
NS = 16
NW = NC * NS
EPW = E // NW
ECH = 125
NCH = EPW // ECH
RPW = 624
RLAST = N - RPW * (NS - 1)

_MESH = plsc.VectorSubcoreMesh(core_axis_name="c", subcore_axis_name="s")
_SC_PARAMS = pltpu.CompilerParams(use_tc_tiling_on_sc=False)


@functools.partial(
    pl.kernel,
    mesh=_MESH,
    compiler_params=_SC_PARAMS,
    out_type=jax.ShapeDtypeStruct((NC, N), jnp.float32),
    scratch_types=[
        pltpu.VMEM((NCH, ECH), jnp.int32),
        pltpu.VMEM((128,), jnp.float32),
        pltpu.VMEM_SHARED((N,), jnp.float32),
    ],
)
def _deg_kernel(dstw_hbm, zeros_hbm, out_hbm, idx_v, ones_v, acc):
    c = lax.axis_index("c")
    s = lax.axis_index("s")
    wid = s * NC + c
    for i in range(8):
        ones_v[pl.ds(i * 16, 16)] = jnp.ones((16,), jnp.float32)

    @pl.when(s == 0)
    def _():
        pltpu.sync_copy(zeros_hbm, acc)

    pltpu.sync_copy(dstw_hbm.at[wid], idx_v)
    plsc.subcore_barrier()

    def body(j, carry):
        pltpu.sync_copy(ones_v.at[pl.ds(0, ECH)], acc.at[idx_v.at[j]], add=True)
        return carry

    lax.fori_loop(0, NCH, body, 0)
    plsc.subcore_barrier()

    @pl.when(s == 0)
    def _():
        pltpu.sync_copy(acc, out_hbm.at[c])


@functools.partial(
    pl.kernel,
    mesh=_MESH,
    compiler_params=_SC_PARAMS,
    out_type=[
        jax.ShapeDtypeStruct((NC, N, DH), jnp.float32),
        jax.ShapeDtypeStruct((NC, N, DH), jnp.float32),
    ],
    scratch_types=[
        pltpu.VMEM((NCH, ECH), jnp.int32),
        pltpu.VMEM((NCH, ECH), jnp.int32),
        pltpu.VMEM((ECH, DH), jnp.float32),
        pltpu.VMEM((ECH, DH), jnp.float32),
        pltpu.VMEM_SHARED((N, DH), jnp.float32),
        pltpu.SemaphoreType.DMA,
        pltpu.SemaphoreType.DMA,
    ],
)
def _agg_kernel(hpa_hbm, hpb_hbm, srcw_hbm, dstw_hbm, zeros_hbm,
                outa_hbm, outb_hbm,
                src_v, dst_v, rows0, rows1, acc, sem0, sem1):
    c = lax.axis_index("c")
    s = lax.axis_index("s")
    wid = s * NC + c
    pltpu.sync_copy(srcw_hbm.at[wid], src_v)
    pltpu.sync_copy(dstw_hbm.at[wid], dst_v)

    for hp_hbm, out_hbm in ((hpa_hbm, outa_hbm), (hpb_hbm, outb_hbm)):
        @pl.when(s < NS - 1)
        def _():
            pltpu.sync_copy(zeros_hbm.at[pl.ds(s * RPW, RPW)],
                            acc.at[pl.ds(s * RPW, RPW)])

        @pl.when(s == NS - 1)
        def _():
            pltpu.sync_copy(zeros_hbm.at[pl.ds((NS - 1) * RPW, RLAST)],
                            acc.at[pl.ds((NS - 1) * RPW, RLAST)])

        plsc.subcore_barrier()

        pltpu.async_copy(hp_hbm.at[src_v.at[0]], rows0, sem0)

        def body(i, carry):
            j = 2 * i
            pltpu.async_copy(hp_hbm.at[src_v.at[j + 1]], rows1, sem1)
            pltpu.make_async_copy(hp_hbm.at[src_v.at[j]], rows0, sem0).wait()
            pltpu.sync_copy(rows0, acc.at[dst_v.at[j]], add=True)

            @pl.when(j + 2 < NCH)
            def _():
                pltpu.async_copy(hp_hbm.at[src_v.at[j + 2]], rows0, sem0)

            pltpu.make_async_copy(hp_hbm.at[src_v.at[j + 1]], rows1, sem1).wait()
            pltpu.sync_copy(rows1, acc.at[dst_v.at[j + 1]], add=True)
            return carry

        lax.fori_loop(0, NCH // 2, body, 0)
        plsc.subcore_barrier()

        @pl.when(s < NS - 1)
        def _():
            pltpu.sync_copy(acc.at[pl.ds(s * RPW, RPW)],
                            out_hbm.at[c, pl.ds(s * RPW, RPW)])

        @pl.when(s == NS - 1)
        def _():
            pltpu.sync_copy(acc.at[pl.ds((NS - 1) * RPW, RLAST)],
                            out_hbm.at[c, pl.ds((NS - 1) * RPW, RLAST)])

        plsc.subcore_barrier()


BN = 2000


def _ln_in(t, g, b):
    m = jnp.mean(t, axis=-1, keepdims=True)
    v = jnp.mean((t - m) * (t - m), axis=-1, keepdims=True)
    return (t - m) * lax.rsqrt(v + 1e-5) * g + b


def _silu(t):
    return t * jax.nn.sigmoid(t)


_ROW = lambda d: pl.BlockSpec((BN, d), lambda i: (i, 0))
_VEC = lambda d: pl.BlockSpec((1, d), lambda i: (0, 0))
_FULL = lambda a, b: pl.BlockSpec((a, b), lambda i: (0, 0))
_AGG = pl.BlockSpec((NC, BN, DH), lambda i: (0, i, 0))


def _mm_scale(x, W, dinv_col):
    def body(x_ref, w_ref, dinv_ref, outa_ref, outb_ref):
        res = jnp.dot(x_ref[...], w_ref[...],
                      preferred_element_type=jnp.float32) * dinv_ref[...]
        outa_ref[...] = res[:, :DH]
        outb_ref[...] = res[:, DH:]
    return pl.pallas_call(
        body,
        grid=(N // BN,),
        in_specs=[_ROW(D), _FULL(D, D), _ROW(1)],
        out_specs=[_ROW(DH), _ROW(DH)],
        out_shape=[
            jax.ShapeDtypeStruct((N, DH), jnp.float32),
            jax.ShapeDtypeStruct((N, DH), jnp.float32),
        ],
    )(x, W, dinv_col)


def _stage(agga, aggb, hpa, hpb, dinv_col, b, g, be, x, W_next):
    def body(agga_ref, aggb_ref, hpa_ref, hpb_ref, dinv_ref, b_ref, g_ref,
             be_ref, x_ref, w_ref, x1_ref, hp2a_ref, hp2b_ref):
        agg = jnp.concatenate(
            [agga_ref[0] + agga_ref[1] + hpa_ref[...],
             aggb_ref[0] + aggb_ref[1] + hpb_ref[...]], axis=-1)
        t = agg * dinv_ref[...] + b_ref[...]
        t = _silu(t)
        x1 = _ln_in(t, g_ref[...], be_ref[...]) + x_ref[...]
        x1_ref[...] = x1
        hp2 = jnp.dot(x1, w_ref[...],
                      preferred_element_type=jnp.float32) * dinv_ref[...]
        hp2a_ref[...] = hp2[:, :DH]
        hp2b_ref[...] = hp2[:, DH:]

    return pl.pallas_call(
        body,
        grid=(N // BN,),
        in_specs=[_AGG, _AGG, _ROW(DH), _ROW(DH), _ROW(1),
                  _VEC(D), _VEC(D), _VEC(D), _ROW(D), _FULL(D, D)],
        out_specs=[_ROW(D), _ROW(DH), _ROW(DH)],
        out_shape=[
            jax.ShapeDtypeStruct((N, D), jnp.float32),
            jax.ShapeDtypeStruct((N, DH), jnp.float32),
            jax.ShapeDtypeStruct((N, DH), jnp.float32),
        ],
    )(agga, aggb, hpa, hpb, dinv_col, b, g, be, x, W_next)


def _final(agga, aggb, hpa, hpb, dinv_col, b, g, be, x,
           cg0, cb0, w1, b1, g1, be1, w2, b2, g2, be2, w3, b3, labels):
    def body(agga_ref, aggb_ref, hpa_ref, hpb_ref, dinv_ref, b_ref, g_ref,
             be_ref, x_ref,
             cg0_ref, cb0_ref, w1_ref, b1_ref, g1_ref, be1_ref,
             w2_ref, b2_ref, g2_ref, be2_ref, w3_ref, b3_ref, lab_ref,
             logits_ref, preds_ref, loss_ref, tp_ref, fp_ref, tn_ref, fn_ref):
        agg = jnp.concatenate(
            [agga_ref[0] + agga_ref[1] + hpa_ref[...],
             aggb_ref[0] + aggb_ref[1] + hpb_ref[...]], axis=-1)
        t = agg * dinv_ref[...] + b_ref[...]
        t = _silu(t)
        x2 = _ln_in(t, g_ref[...], be_ref[...]) + x_ref[...]
        z = _ln_in(x2, cg0_ref[...], cb0_ref[...])
        z = _silu(jnp.dot(z, w1_ref[...], preferred_element_type=jnp.float32)
                  + b1_ref[...])
        z = _ln_in(z, g1_ref[...], be1_ref[...])
        z = _silu(jnp.dot(z, w2_ref[...], preferred_element_type=jnp.float32)
                  + b2_ref[...])
        z = _ln_in(z, g2_ref[...], be2_ref[...])
        logits = jnp.dot(z, w3_ref[...], preferred_element_type=jnp.float32) + b3_ref[...]
        logits_ref[...] = logits

        m = jnp.max(logits, axis=-1, keepdims=True)
        lse = m + jnp.log(jnp.sum(jnp.exp(logits - m), axis=-1, keepdims=True))
        lab = lab_ref[...]
        iot = lax.broadcasted_iota(jnp.int32, logits.shape, 1)
        picked = jnp.sum(jnp.where(iot == lab, logits, 0.0), axis=-1, keepdims=True)
        l0 = jnp.sum(jnp.where(iot == 0, logits, 0.0), axis=-1, keepdims=True)
        l1 = jnp.sum(jnp.where(iot == 1, logits, 0.0), axis=-1, keepdims=True)
        pred = jnp.where(l1 > l0, 1, 0).astype(jnp.int32)
        preds_ref[...] = pred

        p0 = pred == 0
        y0 = lab == 0
        red = lambda a: jnp.sum(a, axis=(0, 1), keepdims=True)
        loss_blk = red(lse - picked)
        tp_blk = red((p0 & y0).astype(jnp.int32))
        fp_blk = red((p0 & (~y0)).astype(jnp.int32))
        tn_blk = red(((~p0) & (~y0)).astype(jnp.int32))
        fn_blk = red(((~p0) & y0).astype(jnp.int32))

        @pl.when(pl.program_id(0) == 0)
        def _():
            loss_ref[...] = jnp.zeros((1, 1), jnp.float32)
            tp_ref[...] = jnp.zeros((1, 1), jnp.int32)
            fp_ref[...] = jnp.zeros((1, 1), jnp.int32)
            tn_ref[...] = jnp.zeros((1, 1), jnp.int32)
            fn_ref[...] = jnp.zeros((1, 1), jnp.int32)

        loss_ref[...] = loss_ref[...] + loss_blk
        tp_ref[...] = tp_ref[...] + tp_blk
        fp_ref[...] = fp_ref[...] + fp_blk
        tn_ref[...] = tn_ref[...] + tn_blk
        fn_ref[...] = fn_ref[...] + fn_blk

    scl = pl.BlockSpec((1, 1), lambda i: (0, 0))
    return pl.pallas_call(
        body,
        grid=(N // BN,),
        in_specs=[_AGG, _AGG, _ROW(DH), _ROW(DH), _ROW(1),
                  _VEC(D), _VEC(D), _VEC(D), _ROW(D),
                  _VEC(D), _VEC(D),
                  _FULL(D, 32), _VEC(32), _VEC(32), _VEC(32),
                  _FULL(32, 8), _VEC(8), _VEC(8), _VEC(8),
                  _FULL(8, 2), _VEC(2),
                  _ROW(1)],
        out_specs=[_ROW(2), _ROW(1), scl, scl, scl, scl, scl],
        out_shape=[
            jax.ShapeDtypeStruct((N, 2), jnp.float32),
            jax.ShapeDtypeStruct((N, 1), jnp.int32),
            jax.ShapeDtypeStruct((1, 1), jnp.float32),
            jax.ShapeDtypeStruct((1, 1), jnp.int32),
            jax.ShapeDtypeStruct((1, 1), jnp.int32),
            jax.ShapeDtypeStruct((1, 1), jnp.int32),
            jax.ShapeDtypeStruct((1, 1), jnp.int32),
        ],
    )(agga, aggb, hpa, hpb, dinv_col, b, g, be, x, cg0, cb0, w1, b1, g1, be1,
      w2, b2, g2, be2, w3, b3, labels)


def kernel(x, edge_index, batch, denoising_label,
           W_g1, b_g1, ln1_g, ln1_b, W_g2, b_g2, ln2_g, ln2_b,
           cls_ln0_g, cls_ln0_b, cls_W1, cls_b1, cls_ln1_g, cls_ln1_b,
           cls_W2, cls_b2, cls_ln2_g, cls_ln2_b, cls_W3, cls_b3):
    src = edge_index[0]
    dst = edge_index[1]
    srcw = src.reshape(NW, NCH, ECH)
    dstw = dst.reshape(NW, NCH, ECH)
    zeros_n = jnp.zeros((N,), jnp.float32)
    zeros_nh = jnp.zeros((N, DH), jnp.float32)

    degp = _deg_kernel(dstw, zeros_n)
    dinv_col = lax.rsqrt(degp[0] + degp[1] + 1.0)[:, None]

    hp1a, hp1b = _mm_scale(x, W_g1, dinv_col)
    agg1a, agg1b = _agg_kernel(hp1a, hp1b, srcw, dstw, zeros_nh)
    x1, hp2a, hp2b = _stage(agg1a, agg1b, hp1a, hp1b, dinv_col, b_g1[None, :],
                            ln1_g[None, :], ln1_b[None, :], x, W_g2)
    agg2a, agg2b = _agg_kernel(hp2a, hp2b, srcw, dstw, zeros_nh)

    logits, preds2, loss_sum, tp2, fp2, tn2, fn2 = _final(
        agg2a, agg2b, hp2a, hp2b, dinv_col, b_g2[None, :], ln2_g[None, :],
        ln2_b[None, :], x1,
        cls_ln0_g[None, :], cls_ln0_b[None, :],
        cls_W1, cls_b1[None, :], cls_ln1_g[None, :], cls_ln1_b[None, :],
        cls_W2, cls_b2[None, :], cls_ln2_g[None, :], cls_ln2_b[None, :],
        cls_W3, cls_b3[None, :],
        denoising_label.astype(jnp.int32)[:, None])

    loss = loss_sum[0, 0] / jnp.float32(N)
    tp = tp2[0, 0]
    fp = fp2[0, 0]
    tn = tn2[0, 0]
    fn = fn2[0, 0]
    total = tp + fp + tn + fn
    acc = (tp + tn) / total
    eps = jnp.finfo(jnp.float32).eps
    snr = 10.0 * jnp.log10(tp.astype(jnp.float32) / (fp.astype(jnp.float32) + eps) + eps)
    recall = tp.astype(jnp.float32) / (tp.astype(jnp.float32) + fn.astype(jnp.float32) + eps)
    valid_snr = jnp.where(recall < 0.2, jnp.float32(-100.0), snr)
    preds = preds2[:, 0]
    return (loss, acc, logits, preds, snr, recall, valid_snr, tp, fp, fn, tn)
